# baseline (speedup 1.0000x reference)
import functools

import jax
import jax.numpy as jnp
from jax.experimental import pallas as pl
from jax.experimental.pallas import tpu as pltpu

_DROP_PROB = 0.2
_MIN_KEEP = 1
_EPS = 1e-8


def _mask_body(imp_ref, noise_ref, km_ref, val_ref, p_ref, *, n_routes,
               n_keep, scale):
    s = [1.0 / (imp_ref[i] + _EPS) + noise_ref[i] for i in range(n_routes)]
    keeps = []
    for i in range(n_routes):
        rank = jnp.int32(0)
        for j in range(n_routes):
            if j < i:
                rank += (s[j] <= s[i]).astype(jnp.int32)
            elif j > i:
                rank += (s[j] < s[i]).astype(jnp.int32)
        keep = rank < n_keep
        keeps.append(keep)
        km_ref[i] = keep.astype(jnp.float32)
        val_ref[i] = jnp.where(keep, jnp.float32(scale), jnp.float32(0.0))
    last = jnp.int32(0)
    for i in range(n_routes):
        last = jnp.where(keeps[i], jnp.int32(i), last)
        p_ref[i] = last


def _compute_mask(importance, noise, n_routes, n_keep, scale):
    body = functools.partial(_mask_body, n_routes=n_routes, n_keep=n_keep,
                             scale=scale)
    return pl.pallas_call(
        body,
        in_specs=[pl.BlockSpec(memory_space=pltpu.SMEM),
                  pl.BlockSpec(memory_space=pltpu.SMEM)],
        out_specs=[pl.BlockSpec(memory_space=pltpu.SMEM)] * 3,
        out_shape=[jax.ShapeDtypeStruct((n_routes,), jnp.float32),
                   jax.ShapeDtypeStruct((n_routes,), jnp.float32),
                   jax.ShapeDtypeStruct((n_routes,), jnp.int32)],
    )(importance, noise)


def _scale_body(p_ref, val_ref, x_ref, o_ref):
    del p_ref
    r = pl.program_id(1)
    o_ref[...] = x_ref[...] * val_ref[r]


def kernel(x, importance):
    n_tokens, n_routes, d = x.shape
    n_keep = max(_MIN_KEEP, int(n_routes * (1.0 - _DROP_PROB)))
    scale = n_routes / float(n_keep)
    noise = jax.random.uniform(jax.random.key(42), (n_routes,),
                               importance.dtype) * 0.5

    km, val, p = _compute_mask(importance, noise, n_routes, n_keep, scale)

    block_t = 512
    nb = n_tokens // block_t
    sub = d // 128
    x4 = x.reshape(n_tokens, n_routes, sub, 128)
    grid_spec = pltpu.PrefetchScalarGridSpec(
        num_scalar_prefetch=2,
        grid=(nb, n_routes),
        in_specs=[pl.BlockSpec((block_t, 1, sub, 128),
                               lambda b, r, p_ref, val_ref: (b, p_ref[r], 0, 0))],
        out_specs=pl.BlockSpec((block_t, 1, sub, 128),
                               lambda b, r, p_ref, val_ref: (b, r, 0, 0)),
    )
    out = pl.pallas_call(
        _scale_body,
        grid_spec=grid_spec,
        out_shape=jax.ShapeDtypeStruct((n_tokens, n_routes, sub, 128),
                                       jnp.float32),
        compiler_params=pltpu.CompilerParams(
            dimension_semantics=("parallel", "arbitrary")),
    )(p, val, x4)
    return out.reshape(n_tokens, n_routes, d), km


# speedup vs baseline: 3.8756x; 3.8756x over previous
import functools

import jax
import jax.numpy as jnp
from jax.experimental import pallas as pl
from jax.experimental.pallas import tpu as pltpu

_DROP_PROB = 0.2
_MIN_KEEP = 1
_EPS = 1e-8


def _mask_body(imp_ref, noise_ref, km_ref, val_ref, p_ref, *, n_routes,
               n_keep, scale):
    s = [1.0 / (imp_ref[i] + _EPS) + noise_ref[i] for i in range(n_routes)]
    keeps = []
    for i in range(n_routes):
        rank = jnp.int32(0)
        for j in range(n_routes):
            if j < i:
                rank += (s[j] <= s[i]).astype(jnp.int32)
            elif j > i:
                rank += (s[j] < s[i]).astype(jnp.int32)
        keep = rank < n_keep
        keeps.append(keep)
        km_ref[i] = keep.astype(jnp.float32)
        val_ref[i] = jnp.where(keep, jnp.float32(scale), jnp.float32(0.0))
    last = jnp.int32(0)
    for i in range(n_routes):
        last = jnp.where(keeps[i], jnp.int32(i), last)
        p_ref[i] = last


def _compute_mask(importance, noise, n_routes, n_keep, scale):
    body = functools.partial(_mask_body, n_routes=n_routes, n_keep=n_keep,
                             scale=scale)
    return pl.pallas_call(
        body,
        in_specs=[pl.BlockSpec(memory_space=pltpu.SMEM),
                  pl.BlockSpec(memory_space=pltpu.SMEM)],
        out_specs=[pl.BlockSpec(memory_space=pltpu.SMEM)] * 3,
        out_shape=[jax.ShapeDtypeStruct((n_routes,), jnp.float32),
                   jax.ShapeDtypeStruct((n_routes,), jnp.float32),
                   jax.ShapeDtypeStruct((n_routes,), jnp.int32)],
    )(importance, noise)


def _scale_body(val_ref, x_ref, o_ref, *, n_routes):
    for r in range(n_routes):
        o_ref[:, r, :] = x_ref[:, r, :] * val_ref[r]


def kernel(x, importance):
    n_tokens, n_routes, d = x.shape
    n_keep = max(_MIN_KEEP, int(n_routes * (1.0 - _DROP_PROB)))
    scale = n_routes / float(n_keep)
    noise = jax.random.uniform(jax.random.key(42), (n_routes,),
                               importance.dtype) * 0.5

    km, val, p = _compute_mask(importance, noise, n_routes, n_keep, scale)

    del p
    block_t = 128
    nb = n_tokens // block_t
    out = pl.pallas_call(
        functools.partial(_scale_body, n_routes=n_routes),
        grid=(nb,),
        in_specs=[pl.BlockSpec(memory_space=pltpu.SMEM),
                  pl.BlockSpec((block_t, n_routes, d), lambda b: (b, 0, 0))],
        out_specs=pl.BlockSpec((block_t, n_routes, d), lambda b: (b, 0, 0)),
        out_shape=jax.ShapeDtypeStruct((n_tokens, n_routes, d), jnp.float32),
        compiler_params=pltpu.CompilerParams(
            dimension_semantics=("parallel",)),
    )(val, x)
    return out, km


# speedup vs baseline: 4.2646x; 1.1004x over previous
import functools

import jax
import jax.numpy as jnp
from jax.experimental import pallas as pl
from jax.experimental.pallas import tpu as pltpu

_DROP_PROB = 0.2
_MIN_KEEP = 1
_EPS = 1e-8


def _keep_flags(imp_ref, noise_ref, n_routes, n_keep):
    s = [1.0 / (imp_ref[i] + _EPS) + noise_ref[i] for i in range(n_routes)]
    keeps = []
    for i in range(n_routes):
        rank = jnp.int32(0)
        for j in range(n_routes):
            if j < i:
                rank += (s[j] <= s[i]).astype(jnp.int32)
            elif j > i:
                rank += (s[j] < s[i]).astype(jnp.int32)
        keeps.append(rank < n_keep)
    return keeps


def _body(imp_ref, noise_ref, x_hbm, o_ref, km_ref, buf, sem, *,
          n_routes, n_keep, scale, block_t, nb):
    b = pl.program_id(0)
    keeps = _keep_flags(imp_ref, noise_ref, n_routes, n_keep)
    for i in range(n_routes):
        km_ref[i] = keeps[i].astype(jnp.float32)

    def issue(block_idx, slot):
        for r in range(n_routes):
            @pl.when(keeps[r])
            def _():
                pltpu.make_async_copy(
                    x_hbm.at[pl.ds(block_idx * block_t, block_t), r, :],
                    buf.at[slot, :, r, :],
                    sem.at[slot]).start()

    @pl.when(b == 0)
    def _():
        issue(0, 0)

    @pl.when(b + 1 < nb)
    def _():
        issue(b + 1, (b + 1) % 2)

    cur = b % 2
    for r in range(n_routes):
        @pl.when(keeps[r])
        def _():
            pltpu.make_async_copy(
                x_hbm.at[pl.ds(0, block_t), r, :],
                buf.at[cur, :, r, :],
                sem.at[cur]).wait()
    for r in range(n_routes):
        @pl.when(keeps[r])
        def _():
            o_ref[:, r, :] = buf[cur, :, r, :] * jnp.float32(scale)

        @pl.when(jnp.logical_not(keeps[r]))
        def _():
            o_ref[:, r, :] = jnp.zeros((block_t, o_ref.shape[2]),
                                       jnp.float32)


def kernel(x, importance):
    n_tokens, n_routes, d = x.shape
    n_keep = max(_MIN_KEEP, int(n_routes * (1.0 - _DROP_PROB)))
    scale = n_routes / float(n_keep)
    noise = jax.random.uniform(jax.random.key(42), (n_routes,),
                               importance.dtype) * 0.5

    block_t = 128
    nb = n_tokens // block_t
    body = functools.partial(_body, n_routes=n_routes, n_keep=n_keep,
                             scale=scale, block_t=block_t, nb=nb)
    out, km = pl.pallas_call(
        body,
        grid=(nb,),
        in_specs=[pl.BlockSpec(memory_space=pltpu.SMEM),
                  pl.BlockSpec(memory_space=pltpu.SMEM),
                  pl.BlockSpec(memory_space=pl.ANY)],
        out_specs=[pl.BlockSpec((block_t, n_routes, d), lambda b: (b, 0, 0)),
                   pl.BlockSpec(memory_space=pltpu.SMEM)],
        out_shape=[jax.ShapeDtypeStruct((n_tokens, n_routes, d), jnp.float32),
                   jax.ShapeDtypeStruct((n_routes,), jnp.float32)],
        scratch_shapes=[pltpu.VMEM((2, block_t, n_routes, d), jnp.float32),
                        pltpu.SemaphoreType.DMA((2,))],
        compiler_params=pltpu.CompilerParams(
            dimension_semantics=("arbitrary",)),
    )(importance, noise, x)
    return out, km


# speedup vs baseline: 4.2995x; 1.0082x over previous
import functools

import jax
import jax.numpy as jnp
from jax.experimental import pallas as pl
from jax.experimental.pallas import tpu as pltpu

_DROP_PROB = 0.2
_MIN_KEEP = 1
_EPS = 1e-8


def _keep_flags(imp_ref, noise_ref, n_routes, n_keep):
    s = [1.0 / (imp_ref[i] + _EPS) + noise_ref[i] for i in range(n_routes)]
    keeps = []
    for i in range(n_routes):
        rank = jnp.int32(0)
        for j in range(n_routes):
            if j < i:
                rank += (s[j] <= s[i]).astype(jnp.int32)
            elif j > i:
                rank += (s[j] < s[i]).astype(jnp.int32)
        keeps.append(rank < n_keep)
    return keeps


def _body(imp_ref, noise_ref, x_hbm, o_ref, km_ref, buf, sem, kf_ref, *,
          n_routes, n_keep, scale, block_t, nb):
    b = pl.program_id(0)

    @pl.when(b == 0)
    def _():
        flags = _keep_flags(imp_ref, noise_ref, n_routes, n_keep)
        for i in range(n_routes):
            kf_ref[i] = flags[i].astype(jnp.int32)
            km_ref[i] = flags[i].astype(jnp.float32)

    keeps = [kf_ref[i] != 0 for i in range(n_routes)]

    def issue(block_idx, slot):
        for r in range(n_routes):
            @pl.when(keeps[r])
            def _():
                pltpu.make_async_copy(
                    x_hbm.at[pl.ds(block_idx * block_t, block_t), r, :],
                    buf.at[slot, :, r, :],
                    sem.at[slot]).start()

    @pl.when(b == 0)
    def _():
        issue(0, 0)

    @pl.when(b + 1 < nb)
    def _():
        issue(b + 1, (b + 1) % 2)

    cur = b % 2
    for r in range(n_routes):
        @pl.when(keeps[r])
        def _():
            pltpu.make_async_copy(
                x_hbm.at[pl.ds(0, block_t), r, :],
                buf.at[cur, :, r, :],
                sem.at[cur]).wait()
    for r in range(n_routes):
        @pl.when(keeps[r])
        def _():
            o_ref[:, r, :] = buf[cur, :, r, :] * jnp.float32(scale)

        @pl.when(jnp.logical_not(keeps[r]))
        def _():
            o_ref[:, r, :] = jnp.zeros((block_t, o_ref.shape[2]),
                                       jnp.float32)


def kernel(x, importance):
    n_tokens, n_routes, d = x.shape
    n_keep = max(_MIN_KEEP, int(n_routes * (1.0 - _DROP_PROB)))
    scale = n_routes / float(n_keep)
    noise = jax.random.uniform(jax.random.key(42), (n_routes,),
                               importance.dtype) * 0.5

    block_t = 128
    nb = n_tokens // block_t
    body = functools.partial(_body, n_routes=n_routes, n_keep=n_keep,
                             scale=scale, block_t=block_t, nb=nb)
    out, km = pl.pallas_call(
        body,
        grid=(nb,),
        in_specs=[pl.BlockSpec(memory_space=pltpu.SMEM),
                  pl.BlockSpec(memory_space=pltpu.SMEM),
                  pl.BlockSpec(memory_space=pl.ANY)],
        out_specs=[pl.BlockSpec((block_t, n_routes, d), lambda b: (b, 0, 0)),
                   pl.BlockSpec(memory_space=pltpu.SMEM)],
        out_shape=[jax.ShapeDtypeStruct((n_tokens, n_routes, d), jnp.float32),
                   jax.ShapeDtypeStruct((n_routes,), jnp.float32)],
        scratch_shapes=[pltpu.VMEM((2, block_t, n_routes, d), jnp.float32),
                        pltpu.SemaphoreType.DMA((2,)),
                        pltpu.SMEM((n_routes,), jnp.int32)],
        compiler_params=pltpu.CompilerParams(
            dimension_semantics=("arbitrary",)),
    )(importance, noise, x)
    return out, km


# speedup vs baseline: 4.3786x; 1.0184x over previous
import functools

import jax
import jax.numpy as jnp
from jax.experimental import pallas as pl
from jax.experimental.pallas import tpu as pltpu

_DROP_PROB = 0.2
_MIN_KEEP = 1
_EPS = 1e-8


def _keep_flags(imp_ref, noise_ref, n_routes, n_keep):
    s = [1.0 / (imp_ref[i] + _EPS) + noise_ref[i] for i in range(n_routes)]
    keeps = []
    for i in range(n_routes):
        rank = jnp.int32(0)
        for j in range(n_routes):
            if j < i:
                rank += (s[j] <= s[i]).astype(jnp.int32)
            elif j > i:
                rank += (s[j] < s[i]).astype(jnp.int32)
        keeps.append(rank < n_keep)
    return keeps


def _body(imp_ref, noise_ref, x_hbm, o_ref, km_ref, buf, sem, kf_ref, *,
          n_routes, n_keep, scale, block_t, nb):
    b = pl.program_id(0)

    @pl.when(b == 0)
    def _():
        flags = _keep_flags(imp_ref, noise_ref, n_routes, n_keep)
        for i in range(n_routes):
            kf_ref[i] = flags[i].astype(jnp.int32)
            km_ref[i] = flags[i].astype(jnp.float32)

    keeps = [kf_ref[i] != 0 for i in range(n_routes)]

    n_slots = buf.shape[0]

    def issue(block_idx, slot):
        for r in range(n_routes):
            @pl.when(keeps[r])
            def _():
                pltpu.make_async_copy(
                    x_hbm.at[pl.ds(block_idx * block_t, block_t), r, :],
                    buf.at[slot, :, r, :],
                    sem.at[slot]).start()

    @pl.when(b == 0)
    def _():
        for i in range(n_slots - 1):
            issue(i, i)

    @pl.when(b + n_slots - 1 < nb)
    def _():
        issue(b + n_slots - 1, (b + n_slots - 1) % n_slots)

    cur = b % n_slots
    for r in range(n_routes):
        @pl.when(keeps[r])
        def _():
            pltpu.make_async_copy(
                x_hbm.at[pl.ds(0, block_t), r, :],
                buf.at[cur, :, r, :],
                sem.at[cur]).wait()
    for r in range(n_routes):
        @pl.when(keeps[r])
        def _():
            o_ref[:, r, :] = buf[cur, :, r, :] * jnp.float32(scale)

        @pl.when(jnp.logical_not(keeps[r]))
        def _():
            o_ref[:, r, :] = jnp.zeros((block_t, o_ref.shape[2]),
                                       jnp.float32)


def kernel(x, importance):
    n_tokens, n_routes, d = x.shape
    n_keep = max(_MIN_KEEP, int(n_routes * (1.0 - _DROP_PROB)))
    scale = n_routes / float(n_keep)
    noise = jax.random.uniform(jax.random.key(42), (n_routes,),
                               importance.dtype) * 0.5

    block_t = 128
    nb = n_tokens // block_t
    body = functools.partial(_body, n_routes=n_routes, n_keep=n_keep,
                             scale=scale, block_t=block_t, nb=nb)
    out, km = pl.pallas_call(
        body,
        grid=(nb,),
        in_specs=[pl.BlockSpec(memory_space=pltpu.SMEM),
                  pl.BlockSpec(memory_space=pltpu.SMEM),
                  pl.BlockSpec(memory_space=pl.ANY)],
        out_specs=[pl.BlockSpec((block_t, n_routes, d), lambda b: (b, 0, 0)),
                   pl.BlockSpec(memory_space=pltpu.SMEM)],
        out_shape=[jax.ShapeDtypeStruct((n_tokens, n_routes, d), jnp.float32),
                   jax.ShapeDtypeStruct((n_routes,), jnp.float32)],
        scratch_shapes=[pltpu.VMEM((3, block_t, n_routes, d), jnp.float32),
                        pltpu.SemaphoreType.DMA((3,)),
                        pltpu.SMEM((n_routes,), jnp.int32)],
        compiler_params=pltpu.CompilerParams(
            dimension_semantics=("arbitrary",)),
    )(importance, noise, x)
    return out, km
